# baseline (speedup 1.0000x reference)
import functools

import jax
import jax.numpy as jnp
from jax import lax
from jax.experimental import pallas as pl
from jax.experimental.pallas import tpu as pltpu
from jax.experimental.pallas import tpu_sc as plsc

_NC = 2
_NS = 16
_NW = _NC * _NS

_CHUNK = 64
_NB = 4
_NI = 8
_N_PAD = 10240
_ROWS_PER_TILE = _N_PAD // _NS


def _linear_tc(h, W, b):
    n, d_in = h.shape
    d_out = W.shape[0]
    blk = 2000
    assert n % blk == 0

    def body(h_ref, w_ref, b_ref, o_ref):
        o_ref[...] = lax.dot_general(
            h_ref[...], w_ref[...],
            (((1,), (1,)), ((), ())),
            preferred_element_type=jnp.float32,
            precision=lax.Precision.HIGHEST,
        ) + b_ref[...]

    return pl.pallas_call(
        body,
        grid=(n // blk,),
        in_specs=[
            pl.BlockSpec((blk, d_in), lambda i: (i, 0)),
            pl.BlockSpec((d_out, d_in), lambda i: (0, 0)),
            pl.BlockSpec((1, d_out), lambda i: (0, 0)),
        ],
        out_specs=pl.BlockSpec((blk, d_out), lambda i: (i, 0)),
        out_shape=jax.ShapeDtypeStruct((n, d_out), jnp.float32),
    )(h, W, b.reshape(1, d_out))


def _make_sc_agg(cpt, d):
    mesh = plsc.VectorSubcoreMesh(core_axis_name="c", subcore_axis_name="s")

    @functools.partial(
        pl.kernel,
        out_type=[
            jax.ShapeDtypeStruct((_NC * _N_PAD, d), jnp.float32),
            jax.ShapeDtypeStruct((_NC * _N_PAD, 16), jnp.float32),
        ],
        mesh=mesh,
        compiler_params=pltpu.CompilerParams(use_tc_tiling_on_sc=False),
        scratch_types=(
            [pltpu.VMEM((1, _CHUNK), jnp.int32)] * (2 * _NI)
            + [pltpu.VMEM((_CHUNK, d), jnp.float32)] * _NB
            + [
                pltpu.VMEM((_CHUNK, 16), jnp.float32),
                pltpu.VMEM((_CHUNK, 16), jnp.float32),
                pltpu.VMEM_SHARED((_N_PAD, d), jnp.float32),
                pltpu.VMEM_SHARED((_N_PAD, 16), jnp.float32),
            ]
            + [pltpu.SemaphoreType.DMA] * (_NB + 2 * _NI)
        ),
    )
    def sc_agg(hlin_hbm, src_hbm, dst_hbm, acc_out, deg_out, *scr):
        sas = list(scr[0:_NI])
        das = list(scr[_NI:2 * _NI])
        rows = list(scr[2 * _NI:2 * _NI + _NB])
        ones_v, z16_v, acc_sh, deg_sh = scr[2 * _NI + _NB:2 * _NI + _NB + 4]
        p = 2 * _NI + _NB + 4
        gsems = list(scr[p:p + _NB])
        sis = list(scr[p + _NB:p + _NB + _NI])
        dis = list(scr[p + _NB + _NI:p + _NB + 2 * _NI])

        c = lax.axis_index("c")
        s = lax.axis_index("s")
        wid = s * _NC + c
        t0 = wid * cpt

        @pl.loop(0, _CHUNK)
        def _(i):
            ones_v[i, pl.ds(0, 16)] = jnp.ones((16,), jnp.float32)
            z16_v[i, pl.ds(0, 16)] = jnp.zeros((16,), jnp.float32)

            @pl.loop(0, d // 16)
            def _(j):
                rows[0][i, pl.ds(j * 16, 16)] = jnp.zeros((16,), jnp.float32)

        base = s * _ROWS_PER_TILE

        @pl.loop(0, _ROWS_PER_TILE // _CHUNK)
        def _(k):
            pltpu.sync_copy(rows[0],
                            acc_sh.at[pl.ds(base + k * _CHUNK, _CHUNK)])
            pltpu.sync_copy(z16_v,
                            deg_sh.at[pl.ds(base + k * _CHUNK, _CHUNK)])

        plsc.subcore_barrier()

        def idx_start(j, k):
            pltpu.async_copy(src_hbm.at[pl.ds(t0 + j, 1)], sas[k], sis[k])
            pltpu.async_copy(dst_hbm.at[pl.ds(t0 + j, 1)], das[k], dis[k])

        def idx_wait(j, k):
            pltpu.make_async_copy(
                src_hbm.at[pl.ds(t0 + j, 1)], sas[k], sis[k]).wait()
            pltpu.make_async_copy(
                dst_hbm.at[pl.ds(t0 + j, 1)], das[k], dis[k]).wait()

        def gather_start(ki, kr):
            pltpu.async_copy(hlin_hbm.at[sas[ki].at[0]], rows[kr], gsems[kr])

        def gather_wait(ki, kr):
            pltpu.make_async_copy(
                hlin_hbm.at[sas[ki].at[0]], rows[kr], gsems[kr]).wait()

        for k in range(_NI):
            idx_start(k, k)
        for k in range(_NB):
            idx_wait(k, k)
            gather_start(k, k)

        @pl.loop(0, cpt // _NI)
        def _(i):
            j0 = _NI * i
            for k in range(_NI):
                j = j0 + k
                kr = k % _NB
                gather_wait(k, kr)
                pltpu.sync_copy(rows[kr], acc_sh.at[das[k].at[0]], add=True)
                pltpu.sync_copy(ones_v, deg_sh.at[das[k].at[0]], add=True)

                @pl.when(j + _NB < cpt)
                def _():
                    idx_wait(j + _NB, (k + _NB) % _NI)
                    gather_start((k + _NB) % _NI, kr)

                @pl.when(j + _NI < cpt)
                def _():
                    idx_start(j + _NI, k)

        plsc.subcore_barrier()

        out_base = c * _N_PAD + base
        pltpu.sync_copy(acc_sh.at[pl.ds(base, _ROWS_PER_TILE)],
                        acc_out.at[pl.ds(out_base, _ROWS_PER_TILE)])
        pltpu.sync_copy(deg_sh.at[pl.ds(base, _ROWS_PER_TILE)],
                        deg_out.at[pl.ds(out_base, _ROWS_PER_TILE)])

    return sc_agg


def _finalize_tc(acc, deg, n, d):
    blk = 2000
    assert n % blk == 0
    acc3 = acc.reshape(_NC, _N_PAD, d)
    deg3 = deg.reshape(_NC, _N_PAD, 16)

    def body(a_ref, g_ref, o_ref):
        a = a_ref[0] + a_ref[1]
        dsum = g_ref[0, :, 0:1] + g_ref[1, :, 0:1]
        o_ref[...] = a / jnp.maximum(dsum, 1.0)

    return pl.pallas_call(
        body,
        grid=(n // blk,),
        in_specs=[
            pl.BlockSpec((_NC, blk, d), lambda i: (0, i, 0)),
            pl.BlockSpec((_NC, blk, 16), lambda i: (0, i, 0)),
        ],
        out_specs=pl.BlockSpec((blk, d), lambda i: (i, 0)),
        out_shape=jax.ShapeDtypeStruct((n, d), jnp.float32),
    )(acc3, deg3)


def kernel(h, edge_index, W, b):
    n, d_in = h.shape
    d = W.shape[0]
    e = edge_index.shape[1]

    h_lin = _linear_tc(h, W, b)

    chunks = -(-e // _CHUNK)
    cpt = -(-chunks // _NW)
    cpt = -(-cpt // _NI) * _NI
    e_pad = cpt * _NW * _CHUNK
    src = edge_index[0].astype(jnp.int32)
    dst = edge_index[1].astype(jnp.int32)
    pad = e_pad - e
    src_p = jnp.concatenate([src, jnp.zeros((pad,), jnp.int32)])
    dst_p = jnp.concatenate([dst, jnp.full((pad,), _N_PAD - 8, jnp.int32)])
    src2 = src_p.reshape(cpt * _NW, _CHUNK)
    dst2 = dst_p.reshape(cpt * _NW, _CHUNK)

    acc, deg = _make_sc_agg(cpt, d)(h_lin, src2, dst2)
    return _finalize_tc(acc, deg, n, d)


# speedup vs baseline: 1.9848x; 1.9848x over previous
import functools

import jax
import jax.numpy as jnp
from jax import lax
from jax.experimental import pallas as pl
from jax.experimental.pallas import tpu as pltpu
from jax.experimental.pallas import tpu_sc as plsc

_NC = 2
_NS = 16

_CHUNK = 128
_NB = 2
_NI = 4
_N_PAD = 10240
_ROWS_PER_TILE = _N_PAD // _NS
_DH = 64


def _linear_tc(h, W, b):
    n, d_in = h.shape
    d_out = W.shape[0]
    blk = 1024
    assert _N_PAD % blk == 0

    def body(h_ref, w_ref, b_ref, o_ref):
        r = lax.dot_general(
            h_ref[...], w_ref[...],
            (((1,), (1,)), ((), ())),
            preferred_element_type=jnp.float32,
            precision=lax.Precision.HIGHEST,
        ) + b_ref[...]
        o_ref[0] = r[:, :_DH]
        o_ref[1] = r[:, _DH:]

    return pl.pallas_call(
        body,
        grid=(_N_PAD // blk,),
        in_specs=[
            pl.BlockSpec((blk, d_in), lambda i: (i, 0)),
            pl.BlockSpec((d_out, d_in), lambda i: (0, 0)),
            pl.BlockSpec((1, d_out), lambda i: (0, 0)),
        ],
        out_specs=pl.BlockSpec((_NC, blk, _DH), lambda i: (0, i, 0)),
        out_shape=jax.ShapeDtypeStruct((_NC, _N_PAD, _DH), jnp.float32),
    )(h, W, b.reshape(1, d_out))


def _make_sc_agg(cpt):
    mesh = plsc.VectorSubcoreMesh(core_axis_name="c", subcore_axis_name="s")

    @functools.partial(
        pl.kernel,
        out_type=[
            jax.ShapeDtypeStruct((_NC * _N_PAD, _DH), jnp.float32),
            jax.ShapeDtypeStruct((_N_PAD, 16), jnp.float32),
        ],
        mesh=mesh,
        compiler_params=pltpu.CompilerParams(use_tc_tiling_on_sc=False),
        scratch_types=(
            [pltpu.VMEM((1, _CHUNK), jnp.int32)] * (2 * _NI)
            + [pltpu.VMEM((_CHUNK, _DH), jnp.float32)] * _NB
            + [
                pltpu.VMEM((_CHUNK, 16), jnp.float32),
                pltpu.VMEM((_CHUNK, 16), jnp.float32),
                pltpu.VMEM_SHARED((_N_PAD, _DH), jnp.float32),
                pltpu.VMEM_SHARED((_N_PAD, _DH), jnp.float32),
                pltpu.VMEM_SHARED((_N_PAD, 16), jnp.float32),
            ]
            + [pltpu.SemaphoreType.DMA] * (_NB + 2 * _NI)
        ),
    )
    def sc_agg(hlin_hbm, src_hbm, dst_hbm, acc_out, deg_out, *scr):
        sas = list(scr[0:_NI])
        das = list(scr[_NI:2 * _NI])
        rows = list(scr[2 * _NI:2 * _NI + _NB])
        ones_v, z16_v, tab_sh, acc_sh, deg_sh = \
            scr[2 * _NI + _NB:2 * _NI + _NB + 5]
        p = 2 * _NI + _NB + 5
        gsems = list(scr[p:p + _NB])
        sis = list(scr[p + _NB:p + _NB + _NI])
        dis = list(scr[p + _NB + _NI:p + _NB + 2 * _NI])

        c = lax.axis_index("c")
        s = lax.axis_index("s")
        t0 = s * cpt

        @pl.loop(0, _CHUNK)
        def _(i):
            ones_v[i, pl.ds(0, 16)] = jnp.ones((16,), jnp.float32)
            z16_v[i, pl.ds(0, 16)] = jnp.zeros((16,), jnp.float32)

            @pl.loop(0, _DH // 16)
            def _(j):
                rows[0][i, pl.ds(j * 16, 16)] = jnp.zeros((16,), jnp.float32)

        base = s * _ROWS_PER_TILE
        pltpu.sync_copy(hlin_hbm.at[c].at[pl.ds(base, _ROWS_PER_TILE)],
                        tab_sh.at[pl.ds(base, _ROWS_PER_TILE)])

        @pl.loop(0, _ROWS_PER_TILE // _CHUNK)
        def _(k):
            pltpu.sync_copy(rows[0],
                            acc_sh.at[pl.ds(base + k * _CHUNK, _CHUNK)])
            pltpu.sync_copy(z16_v,
                            deg_sh.at[pl.ds(base + k * _CHUNK, _CHUNK)])

        plsc.subcore_barrier()

        def idx_start(j, k):
            pltpu.async_copy(src_hbm.at[pl.ds(t0 + j, 1)], sas[k], sis[k])
            pltpu.async_copy(dst_hbm.at[pl.ds(t0 + j, 1)], das[k], dis[k])

        def idx_wait(j, k):
            pltpu.make_async_copy(
                src_hbm.at[pl.ds(t0 + j, 1)], sas[k], sis[k]).wait()
            pltpu.make_async_copy(
                dst_hbm.at[pl.ds(t0 + j, 1)], das[k], dis[k]).wait()

        def gather_start(ki, kr):
            pltpu.async_copy(tab_sh.at[sas[ki].at[0]], rows[kr], gsems[kr])

        def gather_wait(ki, kr):
            pltpu.make_async_copy(
                tab_sh.at[sas[ki].at[0]], rows[kr], gsems[kr]).wait()

        for k in range(_NI):
            idx_start(k, k)
        for k in range(_NB):
            idx_wait(k, k)
            gather_start(k, k)

        @pl.loop(0, cpt // _NI)
        def _(i):
            j0 = _NI * i
            for k in range(_NI):
                j = j0 + k
                kr = k % _NB
                gather_wait(k, kr)
                pltpu.sync_copy(rows[kr], acc_sh.at[das[k].at[0]], add=True)

                @pl.when(c == 0)
                def _():
                    pltpu.sync_copy(ones_v, deg_sh.at[das[k].at[0]],
                                    add=True)

                @pl.when(j + _NB < cpt)
                def _():
                    idx_wait(j + _NB, (k + _NB) % _NI)
                    gather_start((k + _NB) % _NI, kr)

                @pl.when(j + _NI < cpt)
                def _():
                    idx_start(j + _NI, k)

        plsc.subcore_barrier()

        out_base = c * _N_PAD + base
        pltpu.sync_copy(acc_sh.at[pl.ds(base, _ROWS_PER_TILE)],
                        acc_out.at[pl.ds(out_base, _ROWS_PER_TILE)])

        @pl.when(c == 0)
        def _():
            pltpu.sync_copy(deg_sh.at[pl.ds(base, _ROWS_PER_TILE)],
                            deg_out.at[pl.ds(base, _ROWS_PER_TILE)])

    return sc_agg


def _finalize_tc(acc, deg, n, d):
    blk = 2000
    assert n % blk == 0
    acc3 = acc.reshape(_NC, _N_PAD, _DH)

    def body(a_ref, g_ref, o_ref):
        dsum = jnp.maximum(g_ref[:, 0:1], 1.0)
        o_ref[...] = jnp.concatenate([a_ref[0], a_ref[1]], axis=1) / dsum

    return pl.pallas_call(
        body,
        grid=(n // blk,),
        in_specs=[
            pl.BlockSpec((_NC, blk, _DH), lambda i: (0, i, 0)),
            pl.BlockSpec((blk, 16), lambda i: (i, 0)),
        ],
        out_specs=pl.BlockSpec((blk, d), lambda i: (i, 0)),
        out_shape=jax.ShapeDtypeStruct((n, d), jnp.float32),
    )(acc3, deg)


def kernel(h, edge_index, W, b):
    n, d_in = h.shape
    d = W.shape[0]
    e = edge_index.shape[1]

    h_lin = _linear_tc(h, W, b)

    chunks = -(-e // _CHUNK)
    cpt = -(-chunks // _NS) * _NS // _NS
    cpt = -(-cpt // _NI) * _NI
    e_pad = cpt * _NS * _CHUNK
    src = edge_index[0].astype(jnp.int32)
    dst = edge_index[1].astype(jnp.int32)
    pad = e_pad - e
    src_p = jnp.concatenate([src, jnp.zeros((pad,), jnp.int32)])
    dst_p = jnp.concatenate([dst, jnp.full((pad,), _N_PAD - 8, jnp.int32)])
    src2 = src_p.reshape(cpt * _NS, _CHUNK)
    dst2 = dst_p.reshape(cpt * _NS, _CHUNK)

    acc, deg = _make_sc_agg(cpt)(h_lin, src2, dst2)
    return _finalize_tc(acc, deg, n, d)
